# Initial kernel scaffold; baseline (speedup 1.0000x reference)
import jax
import jax.numpy as jnp
from jax.experimental import pallas as pl


def kernel(x, fc_weight, bias):
    raise NotImplementedError("write your pallas kernel here")


# speedup vs baseline: 1.1403x; 1.1403x over previous
import functools

import jax
import jax.numpy as jnp
from jax import lax
from jax.experimental import pallas as pl
from jax.experimental.pallas import tpu as pltpu
from jax.experimental.pallas import tpu_sc as plsc

NUM_FIELDS = 26
FIELD_SIZE = 100000
BATCH = 16384
TABLE_ROWS = NUM_FIELDS * FIELD_SIZE

NC, NS, L = 2, 16, 16
NW = NC * NS
B_PER_W = BATCH // NW
E_PER_W = B_PER_W * NUM_FIELDS
PERIOD = 13
N_OUTER = E_PER_W // (L * PERIOD)


def _sc_body(x_hbm, w_hbm, b_hbm, out_hbm, idx_v, vals_v, out_v, offs_v,
             bias_v, sem):
    wid = lax.axis_index("s") * NC + lax.axis_index("c")
    ebase = wid * E_PER_W
    obase = wid * B_PER_W

    pltpu.sync_copy(x_hbm.at[pl.ds(ebase, E_PER_W)], idx_v)
    pltpu.sync_copy(b_hbm, bias_v)

    lanes = lax.iota(jnp.int32, L)

    def offs_body(p, _):
        v = ((lanes + p * L) % NUM_FIELDS) * FIELD_SIZE
        offs_v[pl.ds(p * L, L)] = v
        return _
    lax.fori_loop(0, PERIOD, offs_body, None)

    def add_body(j, _):
        base = j * (PERIOD * L)
        for k in range(PERIOD):
            sl = pl.ds(base + k * L, L)
            idx_v[sl] = idx_v[sl] + offs_v[pl.ds(k * L, L)]
        return _
    lax.fori_loop(0, N_OUTER, add_body, None)

    pltpu.async_copy(w_hbm.at[idx_v], vals_v, sem).wait()

    bias16 = bias_v[...]
    stride = lanes * NUM_FIELDS
    def red_body(j, _):
        base = j * (L * NUM_FIELDS)
        acc = bias16
        for f in range(NUM_FIELDS):
            acc = acc + plsc.load_gather(vals_v, [stride + (base + f)])
        out_v[pl.ds(j * L, L)] = acc
        return _
    lax.fori_loop(0, B_PER_W // L, red_body, None)

    pltpu.sync_copy(out_v, out_hbm.at[pl.ds(obase, B_PER_W)])


@jax.jit
def _features_linear(x_flat, w_flat, bias):
    mesh = plsc.VectorSubcoreMesh(core_axis_name="c", subcore_axis_name="s",
                                  num_cores=NC, num_subcores=NS)
    out = pl.kernel(
        _sc_body,
        out_type=jax.ShapeDtypeStruct((BATCH,), jnp.float32),
        mesh=mesh,
        scratch_types=[
            pltpu.VMEM((E_PER_W,), jnp.int32),
            pltpu.VMEM((E_PER_W,), jnp.float32),
            pltpu.VMEM((B_PER_W,), jnp.float32),
            pltpu.VMEM((PERIOD * L,), jnp.int32),
            pltpu.VMEM((L,), jnp.float32),
            pltpu.SemaphoreType.DMA,
        ],
        compiler_params=pltpu.CompilerParams(needs_layout_passes=False),
    )(x_flat, w_flat, bias)
    return out


def kernel(x, fc_weight, bias):
    x_flat = x.reshape(-1)
    w_flat = fc_weight.reshape(-1)
    bias16 = jnp.broadcast_to(bias, (L,))
    out = _features_linear(x_flat, w_flat, bias16)
    return out.reshape(BATCH, 1)


# speedup vs baseline: 3.1194x; 2.7356x over previous
import functools

import jax
import jax.numpy as jnp
from jax import lax
from jax.experimental import pallas as pl
from jax.experimental.pallas import tpu as pltpu
from jax.experimental.pallas import tpu_sc as plsc

NUM_FIELDS = 26
FIELD_SIZE = 100000
BATCH = 16384
W_COLS = 8
TABLE_PAD = 2600960
W_ROWS = TABLE_PAD // W_COLS

NC, NS, L = 2, 16, 16
NW = NC * NS
B_PER_W = BATCH // NW
E_PER_W = B_PER_W * NUM_FIELDS
N_CHUNK = B_PER_W // L
F_HALF = NUM_FIELDS // 2
E_HALF = F_HALF * B_PER_W


def _sc_body(xt_hbm, w_hbm, b_hbm, out_hbm, m_v, row0_v, row1_v, vals_v,
             out_v, bias_v, sem, gsem):
    wid = lax.axis_index("s") * NC + lax.axis_index("c")
    obase = wid * B_PER_W

    copies = [
        pltpu.async_copy(xt_hbm.at[f, pl.ds(obase, B_PER_W)],
                         m_v.at[pl.ds(f * B_PER_W, B_PER_W)], sem)
        for f in range(NUM_FIELDS)
    ]
    pltpu.sync_copy(b_hbm, bias_v)
    for c in copies:
        c.wait()

    for f in range(NUM_FIELDS):
        off = f * FIELD_SIZE
        base = f * B_PER_W
        half_v = row0_v if f < F_HALF else row1_v
        hbase = base - (0 if f < F_HALF else E_HALF)

        def add_body(j, _, off=off, base=base, half_v=half_v, hbase=hbase):
            sl = pl.ds(base + j * L, L)
            idx = m_v[sl] + off
            half_v[pl.ds(hbase + j * L, L)] = idx >> 3
            m_v[sl] = idx & 7
            return _
        lax.fori_loop(0, N_CHUNK, add_body, None)

    bias16 = bias_v[...]
    lanes = lax.iota(jnp.int32, L)

    for h, half_v in enumerate((row0_v, row1_v)):
        pltpu.async_copy(w_hbm.at[half_v], vals_v, gsem).wait()

        def red_body(j, _, h=h):
            acc = out_v[pl.ds(j * L, L)] if h else bias16
            for fl in range(F_HALF):
                ebase = fl * B_PER_W + j * L
                pvec = lanes + ebase
                mvec = m_v[pl.ds(h * E_HALF + ebase, L)]
                acc = acc + plsc.load_gather(vals_v, [pvec, mvec])
            out_v[pl.ds(j * L, L)] = acc
            return _
        lax.fori_loop(0, N_CHUNK, red_body, None)

    pltpu.sync_copy(out_v, out_hbm.at[pl.ds(obase, B_PER_W)])


@jax.jit
def _features_linear(xt, w8, bias16):
    mesh = plsc.VectorSubcoreMesh(core_axis_name="c", subcore_axis_name="s",
                                  num_cores=NC, num_subcores=NS)
    out = pl.kernel(
        _sc_body,
        out_type=jax.ShapeDtypeStruct((BATCH,), jnp.float32),
        mesh=mesh,
        scratch_types=[
            pltpu.VMEM((E_PER_W,), jnp.int32),
            pltpu.VMEM((E_HALF,), jnp.int32),
            pltpu.VMEM((E_HALF,), jnp.int32),
            pltpu.VMEM((E_HALF, W_COLS), jnp.float32),
            pltpu.VMEM((B_PER_W,), jnp.float32),
            pltpu.VMEM((L,), jnp.float32),
            pltpu.SemaphoreType.DMA,
            pltpu.SemaphoreType.DMA,
        ],
        compiler_params=pltpu.CompilerParams(
            needs_layout_passes=False,
            use_tc_tiling_on_sc=False,
        ),
    )(xt, w8, bias16)
    return out


def kernel(x, fc_weight, bias):
    xt = x.T
    w8 = jnp.pad(fc_weight, ((0, TABLE_PAD - NUM_FIELDS * FIELD_SIZE), (0, 0))
                 ).reshape(W_ROWS, W_COLS)
    bias16 = jnp.broadcast_to(bias, (L,))
    out = _features_linear(xt, w8, bias16)
    return out.reshape(BATCH, 1)


# speedup vs baseline: 3.2811x; 1.0518x over previous
import functools

import jax
import jax.numpy as jnp
from jax import lax
from jax.experimental import pallas as pl
from jax.experimental.pallas import tpu as pltpu
from jax.experimental.pallas import tpu_sc as plsc

NUM_FIELDS = 26
FIELD_SIZE = 100000
BATCH = 16384
TABLE_PAD = 2600960

NC, NS, L = 2, 16, 16
NW = NC * NS
B_PER_W = BATCH // NW
E_PER_W = B_PER_W * NUM_FIELDS
N_CHUNK = B_PER_W // L


def _sc_body(xt_hbm, w_hbm, b_hbm, out_hbm, idx_v, vals_v, out_v, bias_v,
             sem, gsem):
    wid = lax.axis_index("s") * NC + lax.axis_index("c")
    obase = wid * B_PER_W

    copies = [
        pltpu.async_copy(xt_hbm.at[f, pl.ds(obase, B_PER_W)],
                         idx_v.at[pl.ds(f * B_PER_W, B_PER_W)], sem)
        for f in range(NUM_FIELDS)
    ]
    pltpu.sync_copy(b_hbm, bias_v)
    for c in copies:
        c.wait()

    for f in range(1, NUM_FIELDS):
        off = f * FIELD_SIZE
        base = f * B_PER_W

        def add_body(j, _, off=off, base=base):
            sl = pl.ds(base + j * L, L)
            idx_v[sl] = idx_v[sl] + off
            return _
        lax.fori_loop(0, N_CHUNK, add_body, None)

    pltpu.async_copy(w_hbm.at[idx_v], vals_v, gsem).wait()

    bias16 = bias_v[...]

    def red_body(j, _):
        acc = bias16
        for f in range(NUM_FIELDS):
            acc = acc + vals_v[pl.ds(f * B_PER_W + j * L, L)]
        out_v[pl.ds(j * L, L)] = acc
        return _
    lax.fori_loop(0, N_CHUNK, red_body, None)

    pltpu.sync_copy(out_v, out_hbm.at[pl.ds(obase, B_PER_W)])


@jax.jit
def _features_linear(xt, w1, bias16):
    mesh = plsc.VectorSubcoreMesh(core_axis_name="c", subcore_axis_name="s",
                                  num_cores=NC, num_subcores=NS)
    out = pl.kernel(
        _sc_body,
        out_type=jax.ShapeDtypeStruct((BATCH,), jnp.float32),
        mesh=mesh,
        scratch_types=[
            pltpu.VMEM((E_PER_W,), jnp.int32),
            pltpu.VMEM((E_PER_W,), jnp.float32),
            pltpu.VMEM((B_PER_W,), jnp.float32),
            pltpu.VMEM((L,), jnp.float32),
            pltpu.SemaphoreType.DMA,
            pltpu.SemaphoreType.DMA,
        ],
        compiler_params=pltpu.CompilerParams(
            needs_layout_passes=False,
            use_tc_tiling_on_sc=False,
        ),
    )(xt, w1, bias16)
    return out


def kernel(x, fc_weight, bias):
    xt = x.T
    w1 = jnp.pad(fc_weight, ((0, TABLE_PAD - NUM_FIELDS * FIELD_SIZE), (0, 0))
                 ).reshape(-1)
    bias16 = jnp.broadcast_to(bias, (L,))
    out = _features_linear(xt, w1, bias16)
    return out.reshape(BATCH, 1)


# speedup vs baseline: 3.9748x; 1.2114x over previous
import functools

import jax
import jax.numpy as jnp
from jax import lax
from jax.experimental import pallas as pl
from jax.experimental.pallas import tpu as pltpu
from jax.experimental.pallas import tpu_sc as plsc

NUM_FIELDS = 26
FIELD_SIZE = 100000
BATCH = 16384
TABLE_ROWS = NUM_FIELDS * FIELD_SIZE
MAIN_LIM = 2599936
TAIL = TABLE_ROWS - MAIN_LIM

NC, NS, L = 2, 16, 16
NW = NC * NS
B_PER_W = BATCH // NW
E_PER_W = B_PER_W * NUM_FIELDS
N_CHUNK = B_PER_W // L
LAST = NUM_FIELDS - 1


def _sc_body(xt_hbm, w_hbm, wt_hbm, b_hbm, out_hbm,
             idx_v, side_v, vals_v, out_v, tail_v, bias_v, sem, gsem):
    wid = lax.axis_index("s") * NC + lax.axis_index("c")
    obase = wid * B_PER_W

    copies = [
        pltpu.async_copy(xt_hbm.at[f, pl.ds(obase, B_PER_W)],
                         idx_v.at[pl.ds(f * B_PER_W, B_PER_W)], sem)
        for f in range(NUM_FIELDS)
    ]
    pltpu.sync_copy(b_hbm, bias_v)
    pltpu.sync_copy(wt_hbm, tail_v)
    for c in copies:
        c.wait()

    def add_body(j, _):
        sl25 = pl.ds(LAST * B_PER_W + j * L, L)
        i25 = idx_v[sl25] + LAST * FIELD_SIZE
        side_v[pl.ds(j * L, L)] = i25
        idx_v[sl25] = jnp.minimum(i25, MAIN_LIM - 1)
        for f in range(1, LAST):
            sl = pl.ds(f * B_PER_W + j * L, L)
            idx_v[sl] = idx_v[sl] + f * FIELD_SIZE
        return _
    lax.fori_loop(0, N_CHUNK, add_body, None)

    pltpu.async_copy(w_hbm.at[idx_v], vals_v, gsem).wait()

    bias16 = bias_v[...]

    def red_body(j, _):
        acc = bias16
        for f in range(LAST):
            acc = acc + vals_v[pl.ds(f * B_PER_W + j * L, L)]
        i25 = side_v[pl.ds(j * L, L)]
        t = jnp.maximum(i25 - MAIN_LIM, 0)
        v25 = jnp.where(i25 >= MAIN_LIM,
                        plsc.load_gather(tail_v, [t]),
                        vals_v[pl.ds(LAST * B_PER_W + j * L, L)])
        out_v[pl.ds(j * L, L)] = acc + v25
        return _
    lax.fori_loop(0, N_CHUNK, red_body, None)

    pltpu.sync_copy(out_v, out_hbm.at[pl.ds(obase, B_PER_W)])


@jax.jit
def _features_linear(xt, w1, wt, bias16):
    mesh = plsc.VectorSubcoreMesh(core_axis_name="c", subcore_axis_name="s",
                                  num_cores=NC, num_subcores=NS)
    out = pl.kernel(
        _sc_body,
        out_type=jax.ShapeDtypeStruct((BATCH,), jnp.float32),
        mesh=mesh,
        scratch_types=[
            pltpu.VMEM((E_PER_W,), jnp.int32),
            pltpu.VMEM((B_PER_W,), jnp.int32),
            pltpu.VMEM((E_PER_W,), jnp.float32),
            pltpu.VMEM((B_PER_W,), jnp.float32),
            pltpu.VMEM((TAIL,), jnp.float32),
            pltpu.VMEM((L,), jnp.float32),
            pltpu.SemaphoreType.DMA,
            pltpu.SemaphoreType.DMA,
        ],
        compiler_params=pltpu.CompilerParams(
            needs_layout_passes=False,
            use_tc_tiling_on_sc=False,
        ),
    )(xt, w1, wt, bias16)
    return out


def kernel(x, fc_weight, bias):
    xt = x.T
    w1 = fc_weight[:MAIN_LIM].reshape(-1)
    wt = fc_weight[MAIN_LIM:].reshape(-1)
    bias16 = jnp.broadcast_to(bias, (L,))
    out = _features_linear(xt, w1, wt, bias16)
    return out.reshape(BATCH, 1)


# speedup vs baseline: 4.0342x; 1.0150x over previous
import functools

import jax
import jax.numpy as jnp
from jax import lax
from jax.experimental import pallas as pl
from jax.experimental.pallas import tpu as pltpu
from jax.experimental.pallas import tpu_sc as plsc

NUM_FIELDS = 26
FIELD_SIZE = 100000
BATCH = 16384
TABLE_ROWS = NUM_FIELDS * FIELD_SIZE
MAIN_LIM = 2599936
TAIL = TABLE_ROWS - MAIN_LIM

NC, NS, L = 2, 16, 16
NW = NC * NS
B_PER_W = BATCH // NW
N_CHUNK = B_PER_W // L
F_G = NUM_FIELDS // 2
E_G = F_G * B_PER_W
LAST = NUM_FIELDS - 1


def _sc_body(xt_hbm, w_hbm, wt_hbm, b_hbm, out_hbm,
             idxa_v, idxb_v, valsa_v, valsb_v, side_v, out_v, tail_v,
             bias_v, sema, semb, gsema, gsemb):
    wid = lax.axis_index("s") * NC + lax.axis_index("c")
    obase = wid * B_PER_W

    copies_a = [
        pltpu.async_copy(xt_hbm.at[f, pl.ds(obase, B_PER_W)],
                         idxa_v.at[pl.ds(f * B_PER_W, B_PER_W)], sema)
        for f in range(F_G)
    ]
    copies_b = [
        pltpu.async_copy(xt_hbm.at[F_G + f, pl.ds(obase, B_PER_W)],
                         idxb_v.at[pl.ds(f * B_PER_W, B_PER_W)], semb)
        for f in range(F_G)
    ]
    pltpu.sync_copy(b_hbm, bias_v)
    pltpu.sync_copy(wt_hbm, tail_v)

    for c in copies_a:
        c.wait()

    def add_a(j, _):
        for fl in range(1, F_G):
            sl = pl.ds(fl * B_PER_W + j * L, L)
            idxa_v[sl] = idxa_v[sl] + fl * FIELD_SIZE
        return _
    lax.fori_loop(0, N_CHUNK, add_a, None)
    ga = pltpu.async_copy(w_hbm.at[idxa_v], valsa_v, gsema)

    for c in copies_b:
        c.wait()

    def add_b(j, _):
        sl25 = pl.ds((LAST - F_G) * B_PER_W + j * L, L)
        i25 = idxb_v[sl25] + LAST * FIELD_SIZE
        side_v[pl.ds(j * L, L)] = i25
        idxb_v[sl25] = jnp.minimum(i25, MAIN_LIM - 1)
        for fl in range(F_G - 1):
            sl = pl.ds(fl * B_PER_W + j * L, L)
            idxb_v[sl] = idxb_v[sl] + (F_G + fl) * FIELD_SIZE
        return _
    lax.fori_loop(0, N_CHUNK, add_b, None)
    gb = pltpu.async_copy(w_hbm.at[idxb_v], valsb_v, gsemb)

    bias16 = bias_v[...]
    ga.wait()

    def red_a(j, _):
        acc = bias16
        for fl in range(F_G):
            acc = acc + valsa_v[pl.ds(fl * B_PER_W + j * L, L)]
        out_v[pl.ds(j * L, L)] = acc
        return _
    lax.fori_loop(0, N_CHUNK, red_a, None)

    gb.wait()

    def red_b(j, _):
        acc = out_v[pl.ds(j * L, L)]
        for fl in range(F_G - 1):
            acc = acc + valsb_v[pl.ds(fl * B_PER_W + j * L, L)]
        i25 = side_v[pl.ds(j * L, L)]
        t = jnp.maximum(i25 - MAIN_LIM, 0)
        v25 = jnp.where(i25 >= MAIN_LIM,
                        plsc.load_gather(tail_v, [t]),
                        valsb_v[pl.ds((F_G - 1) * B_PER_W + j * L, L)])
        out_v[pl.ds(j * L, L)] = acc + v25
        return _
    lax.fori_loop(0, N_CHUNK, red_b, None)

    pltpu.sync_copy(out_v, out_hbm.at[pl.ds(obase, B_PER_W)])


@jax.jit
def _features_linear(xt, w1, wt, bias16):
    mesh = plsc.VectorSubcoreMesh(core_axis_name="c", subcore_axis_name="s",
                                  num_cores=NC, num_subcores=NS)
    out = pl.kernel(
        _sc_body,
        out_type=jax.ShapeDtypeStruct((BATCH,), jnp.float32),
        mesh=mesh,
        scratch_types=[
            pltpu.VMEM((E_G,), jnp.int32),
            pltpu.VMEM((E_G,), jnp.int32),
            pltpu.VMEM((E_G,), jnp.float32),
            pltpu.VMEM((E_G,), jnp.float32),
            pltpu.VMEM((B_PER_W,), jnp.int32),
            pltpu.VMEM((B_PER_W,), jnp.float32),
            pltpu.VMEM((TAIL,), jnp.float32),
            pltpu.VMEM((L,), jnp.float32),
            pltpu.SemaphoreType.DMA,
            pltpu.SemaphoreType.DMA,
            pltpu.SemaphoreType.DMA,
            pltpu.SemaphoreType.DMA,
        ],
        compiler_params=pltpu.CompilerParams(
            needs_layout_passes=False,
            use_tc_tiling_on_sc=False,
        ),
    )(xt, w1, wt, bias16)
    return out


def kernel(x, fc_weight, bias):
    xt = x.T
    w1 = fc_weight[:MAIN_LIM].reshape(-1)
    wt = fc_weight[MAIN_LIM:].reshape(-1)
    bias16 = jnp.broadcast_to(bias, (L,))
    out = _features_linear(xt, w1, wt, bias16)
    return out.reshape(BATCH, 1)


# speedup vs baseline: 4.1375x; 1.0256x over previous
import functools

import jax
import jax.numpy as jnp
from jax import lax
from jax.experimental import pallas as pl
from jax.experimental.pallas import tpu as pltpu
from jax.experimental.pallas import tpu_sc as plsc

NUM_FIELDS = 26
FIELD_SIZE = 100000
BATCH = 16384
TABLE_ROWS = NUM_FIELDS * FIELD_SIZE
MAIN_LIM = 2599936
TAIL = TABLE_ROWS - MAIN_LIM

NC, NS, L = 2, 16, 16
NW = NC * NS
B_PER_W = BATCH // NW
N_CHUNK = B_PER_W // L
F_G = NUM_FIELDS // 2
E_G = F_G * B_PER_W
LAST = NUM_FIELDS - 1


def _sc_body(xt_hbm, w_hbm, wt_hbm, b_hbm, out_hbm,
             idxa_v, idxb_v, valsa_v, valsb_v, side_v, out_v, tail_v,
             bias_v, sema, semb, gsema, gsemb):
    wid = lax.axis_index("s") * NC + lax.axis_index("c")
    obase = wid * B_PER_W

    copies_a = [
        pltpu.async_copy(xt_hbm.at[f, pl.ds(obase, B_PER_W)],
                         idxa_v.at[pl.ds(f * B_PER_W, B_PER_W)], sema)
        for f in range(F_G)
    ]
    copies_b = [
        pltpu.async_copy(xt_hbm.at[F_G + f, pl.ds(obase, B_PER_W)],
                         idxb_v.at[pl.ds(f * B_PER_W, B_PER_W)], semb)
        for f in range(F_G)
    ]
    pltpu.sync_copy(b_hbm, bias_v)
    pltpu.sync_copy(wt_hbm, tail_v)

    for c in copies_a:
        c.wait()

    def add_a(j, _):
        for fl in range(1, F_G):
            sl = pl.ds(fl * B_PER_W + j * L, L)
            idxa_v[sl] = idxa_v[sl] + fl * FIELD_SIZE
        return _
    lax.fori_loop(0, N_CHUNK, add_a, None)
    ga = pltpu.async_copy(w_hbm.at[idxa_v], valsa_v, gsema)

    for c in copies_b:
        c.wait()

    def add_b(j, _):
        sl25 = pl.ds((LAST - F_G) * B_PER_W + j * L, L)
        i25 = idxb_v[sl25] + LAST * FIELD_SIZE
        side_v[pl.ds(j * L, L)] = i25
        idxb_v[sl25] = jnp.minimum(i25, MAIN_LIM - 1)
        for fl in range(F_G - 1):
            sl = pl.ds(fl * B_PER_W + j * L, L)
            idxb_v[sl] = idxb_v[sl] + (F_G + fl) * FIELD_SIZE
        return _
    lax.fori_loop(0, N_CHUNK, add_b, None)
    gb = pltpu.async_copy(w_hbm.at[idxb_v], valsb_v, gsemb)

    bias16 = bias_v[...]
    ga.wait()

    def red_a(j, _):
        acc = bias16
        for fl in range(F_G):
            acc = acc + valsa_v[pl.ds(fl * B_PER_W + j * L, L)]
        out_v[pl.ds(j * L, L)] = acc
        return _
    lax.fori_loop(0, N_CHUNK, red_a, None)

    gb.wait()

    def red_b(j, _):
        acc = out_v[pl.ds(j * L, L)]
        for fl in range(F_G - 1):
            acc = acc + valsb_v[pl.ds(fl * B_PER_W + j * L, L)]
        i25 = side_v[pl.ds(j * L, L)]
        t = jnp.maximum(i25 - MAIN_LIM, 0)
        v25 = jnp.where(i25 >= MAIN_LIM,
                        plsc.load_gather(tail_v, [t]),
                        valsb_v[pl.ds((F_G - 1) * B_PER_W + j * L, L)])
        out_v[pl.ds(j * L, L)] = acc + v25
        return _
    lax.fori_loop(0, N_CHUNK, red_b, None)

    pltpu.sync_copy(out_v, out_hbm.at[pl.ds(obase, B_PER_W)])


@jax.jit
def _features_linear(xt, w1, wt, bias16):
    mesh = plsc.VectorSubcoreMesh(core_axis_name="c", subcore_axis_name="s",
                                  num_cores=NC, num_subcores=NS)
    out = pl.kernel(
        _sc_body,
        out_type=jax.ShapeDtypeStruct((BATCH,), jnp.float32),
        mesh=mesh,
        scratch_types=[
            pltpu.VMEM((E_G,), jnp.int32),
            pltpu.VMEM((E_G,), jnp.int32),
            pltpu.VMEM((E_G,), jnp.float32),
            pltpu.VMEM((E_G,), jnp.float32),
            pltpu.VMEM((B_PER_W,), jnp.int32),
            pltpu.VMEM((B_PER_W,), jnp.float32),
            pltpu.VMEM((TAIL,), jnp.float32),
            pltpu.VMEM((L,), jnp.float32),
            pltpu.SemaphoreType.DMA,
            pltpu.SemaphoreType.DMA,
            pltpu.SemaphoreType.DMA,
            pltpu.SemaphoreType.DMA,
        ],
        compiler_params=pltpu.CompilerParams(
            needs_layout_passes=False,
            use_tc_tiling_on_sc=True,
        ),
    )(xt, w1, wt, bias16)
    return out


def kernel(x, fc_weight, bias):
    xt = x.T
    w1 = fc_weight[:MAIN_LIM].reshape(-1)
    wt = fc_weight[MAIN_LIM:].reshape(-1)
    bias16 = jnp.broadcast_to(bias, (L,))
    out = _features_linear(xt, w1, wt, bias16)
    return out.reshape(BATCH, 1)
